# baseline (speedup 1.0000x reference)
import jax
import jax.numpy as jnp
from jax import lax
from jax.experimental import pallas as pl
from jax.experimental.pallas import tpu as pltpu
from jax.experimental.pallas import tpu_sc as plsc

N_IMG, H, W, C = 4, 384, 384, 96
CP = 128
B = 262144
NC, NS, L = 2, 16, 16
NW = NC * NS
BQ = B // NW
K = 64
NCHUNK = BQ // K
CL = C // L


def _body(data_hbm, ii_hbm, ys_hbm, xs_hbm, out_hbm,
          ii_s, ys_s, xs_s,
          idx_v, w_v, rows_v, out_v, gsem, osem):
    wid = lax.axis_index("s") * NC + lax.axis_index("c")
    base = wid * BQ

    pltpu.sync_copy(ii_hbm.at[pl.ds(base, BQ)], ii_s)
    pltpu.sync_copy(ys_hbm.at[pl.ds(base, BQ)], ys_s)
    pltpu.sync_copy(xs_hbm.at[pl.ds(base, BQ)], xs_s)

    def phase_a(ci, p):
        for g in range(K // L):
            s = pl.ds(ci * K + g * L, L)
            d = pl.ds(g * L, L)
            ysv = ys_s[s]
            xsv = xs_s[s]
            iiv = ii_s[s]
            y0 = ysv.astype(jnp.int32)
            x0 = xsv.astype(jnp.int32)
            wy = ysv - y0.astype(jnp.float32)
            wx = xsv - x0.astype(jnp.float32)
            y0 = jnp.minimum(y0, H - 1)
            x0 = jnp.minimum(x0, W - 1)
            y1 = jnp.minimum(y0 + 1, H - 1)
            x1 = jnp.minimum(x0 + 1, W - 1)
            r0 = iiv * (H * W) + y0 * W
            r1 = iiv * (H * W) + y1 * W
            idx_v[p, 0, d] = r0 + x0
            idx_v[p, 1, d] = r0 + x1
            idx_v[p, 2, d] = r1 + x0
            idx_v[p, 3, d] = r1 + x1
            omwy = 1.0 - wy
            omwx = 1.0 - wx
            w_v[p, 0, d] = omwy * omwx
            w_v[p, 1, d] = wy * omwx
            w_v[p, 2, d] = omwy * wx
            w_v[p, 3, d] = wy * wx

    def fire_gathers(p):
        for k in range(4):
            pltpu.async_copy(data_hbm.at[idx_v.at[p, k]], rows_v.at[p, k], gsem)

    def wait_gathers(p):
        for k in range(4):
            pltpu.make_async_copy(data_hbm.at[idx_v.at[p, k]],
                                  rows_v.at[p, k], gsem).wait()

    def compute(ci, p, half):
        def gloop(g, c2):
            gs = pl.ds(g * L, L)
            w0g = w_v[p, 0, gs]
            w1g = w_v[p, 1, gs]
            w2g = w_v[p, 2, gs]
            w3g = w_v[p, 3, gs]
            qvec = lax.iota(jnp.int32, L) + g * L
            for c in range(C):
                cvec = jnp.full((L,), c, jnp.int32)
                v0 = plsc.load_gather(rows_v.at[p, 0], [qvec, cvec])
                v1 = plsc.load_gather(rows_v.at[p, 1], [qvec, cvec])
                v2 = plsc.load_gather(rows_v.at[p, 2], [qvec, cvec])
                v3 = plsc.load_gather(rows_v.at[p, 3], [qvec, cvec])
                out_v[(ci // 2) % 2, c, pl.ds(half * K + g * L, L)] = ((v0 * w0g + v1 * w1g)
                                   + (v2 * w2g + v3 * w3g))
            return c2

        lax.fori_loop(0, K // L, gloop, 0)

    def fire_out_write(ci):
        sb = (ci // 2) % 2
        pltpu.async_copy(out_v.at[sb],
                         out_hbm.at[:, pl.ds(base + (ci // 2) * 2 * K, 2 * K)],
                         osem)

    def wait_out_write(sb):
        pltpu.make_async_copy(out_v.at[sb],
                              out_hbm.at[:, pl.ds(base, 2 * K)], osem).wait()

    phase_a(0, 0)
    fire_gathers(0)

    def pipe(k2, carry):
        for b in range(2):
            i = 2 * k2 + 1 + b
            p = (1 + b) % 2
            q = 1 - p

            @pl.when(i < NCHUNK)
            def _():
                phase_a(i, p)
                fire_gathers(p)

            wait_gathers(q)

            j = i - 1
            jh = j % 2
            @pl.when(jnp.logical_and(jh == 0, j >= 4))
            def _():
                wait_out_write((j // 2) % 2)

            compute(j, q, jh)

            @pl.when(jh == 1)
            def _():
                fire_out_write(j)
        return carry

    lax.fori_loop(0, NCHUNK // 2, pipe, 0)
    wait_out_write(0)
    wait_out_write(1)


_mesh = plsc.VectorSubcoreMesh(core_axis_name="c", subcore_axis_name="s",
                               num_cores=NC, num_subcores=NS)

_sc_call = pl.kernel(
    _body,
    out_type=jax.ShapeDtypeStruct((C, B), jnp.float32),
    mesh=_mesh,
    scratch_types=[
        pltpu.VMEM((BQ,), jnp.int32),
        pltpu.VMEM((BQ,), jnp.float32),
        pltpu.VMEM((BQ,), jnp.float32),
        pltpu.VMEM((2, 4, K), jnp.int32),
        pltpu.VMEM((2, 4, K), jnp.float32),
        pltpu.VMEM((2, 4, K, CP), jnp.float32),
        pltpu.VMEM((2, C, 2 * K), jnp.float32),
        pltpu.SemaphoreType.DMA,
        pltpu.SemaphoreType.DMA,
    ],
    compiler_params=pltpu.CompilerParams(use_tc_tiling_on_sc=True, needs_layout_passes=False),
)


@jax.jit
def kernel(data, iind, ys, xs):
    flat = jnp.pad(data.reshape(N_IMG * H * W, C), ((0, 0), (0, CP - C)))
    ii = iind.astype(jnp.int32)
    return _sc_call(flat, ii, ys, xs).T


# speedup vs baseline: 2.1632x; 2.1632x over previous
import jax
import jax.numpy as jnp
from jax import lax
from jax.experimental import pallas as pl
from jax.experimental.pallas import tpu as pltpu
from jax.experimental.pallas import tpu_sc as plsc

N_IMG, H, W, C = 4, 384, 384, 96
CP = 128
B = 262144
NC, NS, L = 2, 16, 16
NW = NC * NS
BQ = B // NW
K = 64
NCHUNK = BQ // K
CL = C // L


def _body(data_hbm, ii_hbm, ys_hbm, xs_hbm, out_hbm,
          ii_s, ys_s, xs_s,
          idx_v, w_v, rows_v, out_v, gsem, osem):
    wid = lax.axis_index("s") * NC + lax.axis_index("c")
    base = wid * BQ

    pltpu.sync_copy(ii_hbm.at[pl.ds(base, BQ)], ii_s)
    pltpu.sync_copy(ys_hbm.at[pl.ds(base, BQ)], ys_s)
    pltpu.sync_copy(xs_hbm.at[pl.ds(base, BQ)], xs_s)

    def phase_a(ci, p):
        for g in range(K // L):
            s = pl.ds(ci * K + g * L, L)
            d = pl.ds(g * L, L)
            ysv = ys_s[s]
            xsv = xs_s[s]
            iiv = ii_s[s]
            y0 = ysv.astype(jnp.int32)
            x0 = xsv.astype(jnp.int32)
            wy = ysv - y0.astype(jnp.float32)
            wx = xsv - x0.astype(jnp.float32)
            y0 = jnp.minimum(y0, H - 1)
            x0 = jnp.minimum(x0, W - 1)
            y1 = jnp.minimum(y0 + 1, H - 1)
            x1 = jnp.minimum(x0 + 1, W - 1)
            r0 = iiv * (H * W) + y0 * W
            r1 = iiv * (H * W) + y1 * W
            idx_v[p, 0, d] = r0 + x0
            idx_v[p, 1, d] = r0 + x1
            idx_v[p, 2, d] = r1 + x0
            idx_v[p, 3, d] = r1 + x1
            omwy = 1.0 - wy
            omwx = 1.0 - wx
            w_v[p, 0, d] = omwy * omwx
            w_v[p, 1, d] = wy * omwx
            w_v[p, 2, d] = omwy * wx
            w_v[p, 3, d] = wy * wx

    def fire_gathers(p):
        for k in range(4):
            pltpu.async_copy(data_hbm.at[idx_v.at[p, k]], rows_v.at[p, k], gsem)

    def wait_gathers(p):
        for k in range(4):
            pltpu.make_async_copy(data_hbm.at[idx_v.at[p, k]],
                                  rows_v.at[p, k], gsem).wait()

    def compute(ci, p, half):
        sb = (ci // 2) % 2
        cvecs = [lax.iota(jnp.int32, L) + c * L for c in range(CL)]

        def gloop(g, c2):
            gs = pl.ds(g * L, L)
            w0g = w_v[p, 0, gs]
            w1g = w_v[p, 1, gs]
            w2g = w_v[p, 2, gs]
            w3g = w_v[p, 3, gs]
            for j in range(L):
                q = g * L + j
                w0 = jnp.full((L,), w0g[j], jnp.float32)
                w1 = jnp.full((L,), w1g[j], jnp.float32)
                w2 = jnp.full((L,), w2g[j], jnp.float32)
                w3 = jnp.full((L,), w3g[j], jnp.float32)
                qvec = jnp.full((L,), half * K + q, jnp.int32)
                for c in range(CL):
                    cs = pl.ds(c * L, L)
                    v = ((rows_v[p, 0, q, cs] * w0 + rows_v[p, 1, q, cs] * w1)
                         + (rows_v[p, 2, q, cs] * w2 + rows_v[p, 3, q, cs] * w3))
                    plsc.store_scatter(out_v.at[sb], [cvecs[c], qvec], v)
            return c2

        lax.fori_loop(0, K // L, gloop, 0)

    def fire_out_write(ci):
        sb = (ci // 2) % 2
        pltpu.async_copy(out_v.at[sb],
                         out_hbm.at[:, pl.ds(base + (ci // 2) * 2 * K, 2 * K)],
                         osem)

    def wait_out_write(sb):
        pltpu.make_async_copy(out_v.at[sb],
                              out_hbm.at[:, pl.ds(base, 2 * K)], osem).wait()

    phase_a(0, 0)
    fire_gathers(0)

    def pipe(k2, carry):
        for b in range(2):
            i = 2 * k2 + 1 + b
            p = (1 + b) % 2
            q = 1 - p

            @pl.when(i < NCHUNK)
            def _():
                phase_a(i, p)
                fire_gathers(p)

            wait_gathers(q)

            j = i - 1
            jh = j % 2
            @pl.when(jnp.logical_and(jh == 0, j >= 4))
            def _():
                wait_out_write((j // 2) % 2)

            compute(j, q, jh)

            @pl.when(jh == 1)
            def _():
                fire_out_write(j)
        return carry

    lax.fori_loop(0, NCHUNK // 2, pipe, 0)
    wait_out_write(0)
    wait_out_write(1)


_mesh = plsc.VectorSubcoreMesh(core_axis_name="c", subcore_axis_name="s",
                               num_cores=NC, num_subcores=NS)

_sc_call = pl.kernel(
    _body,
    out_type=jax.ShapeDtypeStruct((C, B), jnp.float32),
    mesh=_mesh,
    scratch_types=[
        pltpu.VMEM((BQ,), jnp.int32),
        pltpu.VMEM((BQ,), jnp.float32),
        pltpu.VMEM((BQ,), jnp.float32),
        pltpu.VMEM((2, 4, K), jnp.int32),
        pltpu.VMEM((2, 4, K), jnp.float32),
        pltpu.VMEM((2, 4, K, CP), jnp.float32),
        pltpu.VMEM((2, C, 2 * K), jnp.float32),
        pltpu.SemaphoreType.DMA,
        pltpu.SemaphoreType.DMA,
    ],
    compiler_params=pltpu.CompilerParams(use_tc_tiling_on_sc=True, needs_layout_passes=False),
)


@jax.jit
def kernel(data, iind, ys, xs):
    flat = jnp.pad(data.reshape(N_IMG * H * W, C), ((0, 0), (0, CP - C)))
    ii = iind.astype(jnp.int32)
    return _sc_call(flat, ii, ys, xs).T


# speedup vs baseline: 3.3334x; 1.5410x over previous
import jax
import jax.numpy as jnp
from jax import lax
from jax.experimental import pallas as pl
from jax.experimental.pallas import tpu as pltpu
from jax.experimental.pallas import tpu_sc as plsc

N_IMG, H, W, C = 4, 384, 384, 96
CP = 128
B = 262144
NC, NS, L = 2, 16, 16
NW = NC * NS
BQ = B // NW
K = 64
NCHUNK = BQ // K
CL = C // L


def _body(data_hbm, ii_hbm, ys_hbm, xs_hbm, out_hbm,
          ii_s, ys_s, xs_s,
          idx_v, w_v, rows_v, out_v, gsem, osem):
    wid = lax.axis_index("s") * NC + lax.axis_index("c")
    base = wid * BQ

    pltpu.sync_copy(ii_hbm.at[pl.ds(base, BQ)], ii_s)
    pltpu.sync_copy(ys_hbm.at[pl.ds(base, BQ)], ys_s)
    pltpu.sync_copy(xs_hbm.at[pl.ds(base, BQ)], xs_s)

    def phase_a(ci, p):
        for g in range(K // L):
            s = pl.ds(ci * K + g * L, L)
            d = pl.ds(g * L, L)
            ysv = ys_s[s]
            xsv = xs_s[s]
            iiv = ii_s[s]
            y0 = ysv.astype(jnp.int32)
            x0 = xsv.astype(jnp.int32)
            wy = ysv - y0.astype(jnp.float32)
            wx = xsv - x0.astype(jnp.float32)
            y0 = jnp.minimum(y0, H - 1)
            x0 = jnp.minimum(x0, W - 1)
            y1 = jnp.minimum(y0 + 1, H - 1)
            x1 = jnp.minimum(x0 + 1, W - 1)
            r0 = iiv * (H * W) + y0 * W
            r1 = iiv * (H * W) + y1 * W
            idx_v[p, 0, d] = r0 + x0
            idx_v[p, 1, d] = r0 + x1
            idx_v[p, 2, d] = r1 + x0
            idx_v[p, 3, d] = r1 + x1
            omwy = 1.0 - wy
            omwx = 1.0 - wx
            w_v[p, 0, d] = omwy * omwx
            w_v[p, 1, d] = wy * omwx
            w_v[p, 2, d] = omwy * wx
            w_v[p, 3, d] = wy * wx

    def fire_gathers(p):
        for k in range(4):
            pltpu.async_copy(data_hbm.at[idx_v.at[p, k]], rows_v.at[p, k], gsem)

    def wait_gathers(p):
        for k in range(4):
            pltpu.make_async_copy(data_hbm.at[idx_v.at[p, k]],
                                  rows_v.at[p, k], gsem).wait()

    def compute(ci, p):
        def gloop(g, c2):
            gs = pl.ds(g * L, L)
            w0g = w_v[p, 0, gs]
            w1g = w_v[p, 1, gs]
            w2g = w_v[p, 2, gs]
            w3g = w_v[p, 3, gs]
            for j in range(L):
                q = g * L + j
                w0 = jnp.full((L,), w0g[j], jnp.float32)
                w1 = jnp.full((L,), w1g[j], jnp.float32)
                w2 = jnp.full((L,), w2g[j], jnp.float32)
                w3 = jnp.full((L,), w3g[j], jnp.float32)
                for c in range(CL):
                    cs = pl.ds(c * L, L)
                    out_v[p, q, cs] = (
                        (rows_v[p, 0, q, cs] * w0 + rows_v[p, 1, q, cs] * w1)
                        + (rows_v[p, 2, q, cs] * w2 + rows_v[p, 3, q, cs] * w3))
            return c2

        lax.fori_loop(0, K // L, gloop, 0)
        pltpu.async_copy(out_v.at[p], out_hbm.at[pl.ds(base + ci * K, K)],
                         osem)

    def wait_out_write(p):
        pltpu.make_async_copy(out_v.at[p], out_hbm.at[pl.ds(base, K)],
                              osem).wait()

    phase_a(0, 0)
    fire_gathers(0)

    def pipe(k2, carry):
        for b in range(2):
            i = 2 * k2 + 1 + b
            p = (1 + b) % 2
            q = 1 - p

            @pl.when(i < NCHUNK)
            def _():
                phase_a(i, p)
                fire_gathers(p)

            wait_gathers(q)

            @pl.when(i >= 3)
            def _():
                wait_out_write(q)

            compute(i - 1, q)
        return carry

    lax.fori_loop(0, NCHUNK // 2, pipe, 0)
    wait_out_write(0)
    wait_out_write(1)


_mesh = plsc.VectorSubcoreMesh(core_axis_name="c", subcore_axis_name="s",
                               num_cores=NC, num_subcores=NS)

_sc_call = pl.kernel(
    _body,
    out_type=jax.ShapeDtypeStruct((B, CP), jnp.float32),
    mesh=_mesh,
    scratch_types=[
        pltpu.VMEM((BQ,), jnp.int32),
        pltpu.VMEM((BQ,), jnp.float32),
        pltpu.VMEM((BQ,), jnp.float32),
        pltpu.VMEM((2, 4, K), jnp.int32),
        pltpu.VMEM((2, 4, K), jnp.float32),
        pltpu.VMEM((2, 4, K, CP), jnp.float32),
        pltpu.VMEM((2, K, CP), jnp.float32),
        pltpu.SemaphoreType.DMA,
        pltpu.SemaphoreType.DMA,
    ],
    compiler_params=pltpu.CompilerParams(use_tc_tiling_on_sc=True),
)


@jax.jit
def kernel(data, iind, ys, xs):
    flat = jnp.pad(data.reshape(N_IMG * H * W, C), ((0, 0), (0, CP - C)))
    ii = iind.astype(jnp.int32)
    return _sc_call(flat, ii, ys, xs)[:, :C]


# speedup vs baseline: 3.3365x; 1.0009x over previous
import jax
import jax.numpy as jnp
from jax import lax
from jax.experimental import pallas as pl
from jax.experimental.pallas import tpu as pltpu
from jax.experimental.pallas import tpu_sc as plsc

N_IMG, H, W, C = 4, 384, 384, 96
CP = 128
B = 262144
NC, NS, L = 2, 16, 16
NW = NC * NS
BQ = B // NW
K = 64
NCHUNK = BQ // K
CL = C // L


def _body(data_hbm, ii_hbm, ys_hbm, xs_hbm, out_hbm,
          ii_s, ys_s, xs_s,
          idx_v, w_v, rows_v, out_v, gsem, osem):
    wid = lax.axis_index("s") * NC + lax.axis_index("c")
    base = wid * BQ

    pltpu.sync_copy(ii_hbm.at[pl.ds(base, BQ)], ii_s)
    pltpu.sync_copy(ys_hbm.at[pl.ds(base, BQ)], ys_s)
    pltpu.sync_copy(xs_hbm.at[pl.ds(base, BQ)], xs_s)

    def phase_a(ci, p):
        for g in range(K // L):
            s = pl.ds(ci * K + g * L, L)
            d = pl.ds(g * L, L)
            ysv = ys_s[s]
            xsv = xs_s[s]
            iiv = ii_s[s]
            y0 = ysv.astype(jnp.int32)
            x0 = xsv.astype(jnp.int32)
            wy = ysv - y0.astype(jnp.float32)
            wx = xsv - x0.astype(jnp.float32)
            y0 = jnp.minimum(y0, H - 1)
            x0 = jnp.minimum(x0, W - 1)
            y1 = jnp.minimum(y0 + 1, H - 1)
            x1 = jnp.minimum(x0 + 1, W - 1)
            r0 = iiv * (H * W) + y0 * W
            r1 = iiv * (H * W) + y1 * W
            d2 = pl.ds(K + g * L, L)
            idx_v[p, 0, d] = r0 + x0
            idx_v[p, 0, d2] = r0 + x1
            idx_v[p, 1, d] = r1 + x0
            idx_v[p, 1, d2] = r1 + x1
            omwy = 1.0 - wy
            omwx = 1.0 - wx
            w_v[p, 0, d] = omwy * omwx
            w_v[p, 1, d] = wy * omwx
            w_v[p, 2, d] = omwy * wx
            w_v[p, 3, d] = wy * wx

    def fire_gathers(p):
        for k in range(2):
            pltpu.async_copy(data_hbm.at[idx_v.at[p, k]], rows_v.at[p, k], gsem)

    def wait_gathers(p):
        for k in range(2):
            pltpu.make_async_copy(data_hbm.at[idx_v.at[p, k]],
                                  rows_v.at[p, k], gsem).wait()

    def compute(ci, p):
        def gloop(g, c2):
            gs = pl.ds(g * L, L)
            w0g = w_v[p, 0, gs]
            w1g = w_v[p, 1, gs]
            w2g = w_v[p, 2, gs]
            w3g = w_v[p, 3, gs]
            for j in range(L):
                q = g * L + j
                w0 = jnp.full((L,), w0g[j], jnp.float32)
                w1 = jnp.full((L,), w1g[j], jnp.float32)
                w2 = jnp.full((L,), w2g[j], jnp.float32)
                w3 = jnp.full((L,), w3g[j], jnp.float32)
                for c in range(CL):
                    cs = pl.ds(c * L, L)
                    out_v[p, q, cs] = (
                        (rows_v[p, 0, q, cs] * w0
                         + rows_v[p, 0, K + q, cs] * w1)
                        + (rows_v[p, 1, q, cs] * w2
                           + rows_v[p, 1, K + q, cs] * w3))
            return c2

        lax.fori_loop(0, K // L, gloop, 0)
        pltpu.async_copy(out_v.at[p], out_hbm.at[pl.ds(base + ci * K, K)],
                         osem)

    def wait_out_write(p):
        pltpu.make_async_copy(out_v.at[p], out_hbm.at[pl.ds(base, K)],
                              osem).wait()

    phase_a(0, 0)
    fire_gathers(0)

    def pipe(k2, carry):
        for b in range(2):
            i = 2 * k2 + 1 + b
            p = (1 + b) % 2
            q = 1 - p

            @pl.when(i < NCHUNK)
            def _():
                phase_a(i, p)
                fire_gathers(p)

            wait_gathers(q)

            @pl.when(i >= 3)
            def _():
                wait_out_write(q)

            compute(i - 1, q)
        return carry

    lax.fori_loop(0, NCHUNK // 2, pipe, 0)
    wait_out_write(0)
    wait_out_write(1)


_mesh = plsc.VectorSubcoreMesh(core_axis_name="c", subcore_axis_name="s",
                               num_cores=NC, num_subcores=NS)

_sc_call = pl.kernel(
    _body,
    out_type=jax.ShapeDtypeStruct((B, CP), jnp.float32),
    mesh=_mesh,
    scratch_types=[
        pltpu.VMEM((BQ,), jnp.int32),
        pltpu.VMEM((BQ,), jnp.float32),
        pltpu.VMEM((BQ,), jnp.float32),
        pltpu.VMEM((2, 2, 2 * K), jnp.int32),
        pltpu.VMEM((2, 4, K), jnp.float32),
        pltpu.VMEM((2, 2, 2 * K, CP), jnp.float32),
        pltpu.VMEM((2, K, CP), jnp.float32),
        pltpu.SemaphoreType.DMA,
        pltpu.SemaphoreType.DMA,
    ],
    compiler_params=pltpu.CompilerParams(use_tc_tiling_on_sc=True),
)


@jax.jit
def kernel(data, iind, ys, xs):
    flat = jnp.pad(data.reshape(N_IMG * H * W, C), ((0, 0), (0, CP - C)))
    ii = iind.astype(jnp.int32)
    return _sc_call(flat, ii, ys, xs)[:, :C]
